# baseline (speedup 1.0000x reference)
import functools

import jax
import jax.numpy as jnp
from jax import lax
from jax.experimental import pallas as pl
from jax.experimental.pallas import tpu as pltpu
from jax.experimental.pallas import tpu_sc as plsc

_B = 16384
_D = 128
_NC = 2
_NS = 16
_NW = _NC * _NS
_BPW = _B // _NW

_BS = 1024
_GRID = _B // _BS


def _sc_gather(table, idx):
    mesh = plsc.VectorSubcoreMesh(core_axis_name="c", subcore_axis_name="s")

    @functools.partial(
        pl.kernel,
        mesh=mesh,
        out_type=jax.ShapeDtypeStruct((_B, _D), jnp.float32),
        scratch_types=[
            pltpu.VMEM((_BPW,), jnp.int32),
            pltpu.VMEM((_BPW, _D), jnp.float32),
            pltpu.SemaphoreType.DMA,
        ],
    )
    def k(table_hbm, idx_hbm, out_hbm, idx_v, rows_v, sem):
        wid = lax.axis_index("s") * _NC + lax.axis_index("c")
        base = wid * _BPW
        pltpu.sync_copy(idx_hbm.at[pl.ds(base, _BPW)], idx_v)
        pltpu.async_copy(table_hbm.at[idx_v], rows_v, sem).wait()
        pltpu.sync_copy(rows_v, out_hbm.at[pl.ds(base, _BPW)])

    return k(table, idx)


def _tc_body(xT_ref, idx_ref, epc_ref, W1T_ref, b1_ref,
             EcmsT_ref, EfnfT_ref, W2T_ref, b2_ref, o_ref):
    f1T = jnp.dot(W1T_ref[...], xT_ref[...],
                  preferred_element_type=jnp.float32) + b1_ref[...]
    f1T = jnp.where(f1T >= 0, f1T, 0.01 * f1T)

    idxb = idx_ref[0]
    ohcT = (idxb[0:1, :] == lax.broadcasted_iota(jnp.int32, (4, _BS), 0)
            ).astype(jnp.float32)
    ohfT = (idxb[1:2, :] == lax.broadcasted_iota(jnp.int32, (8, _BS), 0)
            ).astype(jnp.float32)

    W2T = W2T_ref[...]
    TcmsT = jnp.dot(W2T[:, 64:96], EcmsT_ref[...],
                    preferred_element_type=jnp.float32)
    TfnfT = jnp.dot(W2T[:, 96:128], EfnfT_ref[...],
                    preferred_element_type=jnp.float32)

    acc = jnp.dot(W2T[:, 0:64], f1T, preferred_element_type=jnp.float32)
    acc += jnp.dot(TcmsT, ohcT, preferred_element_type=jnp.float32)
    acc += jnp.dot(TfnfT, ohfT, preferred_element_type=jnp.float32)
    acc += lax.dot_general(W2T[:, 128:256], epc_ref[...],
                           (((1,), (1,)), ((), ())),
                           preferred_element_type=jnp.float32)
    o_ref[...] = jnp.tanh(acc + b2_ref[...])


def kernel(x, idx_cms, idx_fnf, idx_pc, W1, b1, E_cms, E_fnf, E_pc, W2, b2):
    e_pc = _sc_gather(E_pc, idx_pc.reshape(_B))

    idx2 = jnp.stack([idx_cms.reshape(_GRID, _BS),
                      idx_fnf.reshape(_GRID, _BS)], axis=1)
    xT = x.T
    W1T = W1.T
    W2T = W2.T
    EcmsT = E_cms.T
    EfnfT = jnp.zeros((32, 8), jnp.float32).at[:, :5].set(E_fnf.T)
    b1c = b1.reshape(64, 1)
    b2c = b2.reshape(64, 1)

    full = lambda i: (0, 0)
    yT = pl.pallas_call(
        _tc_body,
        grid=(_GRID,),
        in_specs=[
            pl.BlockSpec((3, _BS), lambda i: (0, i)),
            pl.BlockSpec((1, 2, _BS), lambda i: (i, 0, 0)),
            pl.BlockSpec((_BS, _D), lambda i: (i, 0)),
            pl.BlockSpec((64, 3), full),
            pl.BlockSpec((64, 1), full),
            pl.BlockSpec((32, 4), full),
            pl.BlockSpec((32, 8), full),
            pl.BlockSpec((64, 256), full),
            pl.BlockSpec((64, 1), full),
        ],
        out_specs=pl.BlockSpec((64, _BS), lambda i: (0, i)),
        out_shape=jax.ShapeDtypeStruct((64, _B), jnp.float32),
    )(xT, idx2, e_pc, W1T, b1c, EcmsT, EfnfT, W2T, b2c)
    return yT.T


# speedup vs baseline: 1.2123x; 1.2123x over previous
import functools

import jax
import jax.numpy as jnp
from jax import lax
from jax.experimental import pallas as pl
from jax.experimental.pallas import tpu as pltpu
from jax.experimental.pallas import tpu_sc as plsc

_B = 16384
_D = 128
_NC = 2
_NS = 16
_NW = _NC * _NS
_BPW = _B // _NW

_BS = 8192
_GRID = _B // _BS


def _sc_gather(table, idx):
    mesh = plsc.VectorSubcoreMesh(core_axis_name="c", subcore_axis_name="s")

    @functools.partial(
        pl.kernel,
        mesh=mesh,
        out_type=jax.ShapeDtypeStruct((_B, _D), jnp.float32),
        scratch_types=[
            pltpu.VMEM((_BPW,), jnp.int32),
            pltpu.VMEM((_BPW, _D), jnp.float32),
            pltpu.SemaphoreType.DMA,
        ],
    )
    def k(table_hbm, idx_hbm, out_hbm, idx_v, rows_v, sem):
        wid = lax.axis_index("s") * _NC + lax.axis_index("c")
        base = wid * _BPW
        pltpu.sync_copy(idx_hbm.at[pl.ds(base, _BPW)], idx_v)
        pltpu.async_copy(table_hbm.at[idx_v], rows_v, sem).wait()
        pltpu.sync_copy(rows_v, out_hbm.at[pl.ds(base, _BPW)])

    return k(table, idx)


def _tc_body(xT_ref, idx_ref, epc_ref, W1T_ref, b1_ref,
             EcmsT_ref, EfnfT_ref, W2T_ref, b2_ref, o_ref):
    f1T = jnp.dot(W1T_ref[...], xT_ref[...],
                  preferred_element_type=jnp.float32) + b1_ref[...]
    f1T = jnp.where(f1T >= 0, f1T, 0.01 * f1T)

    idxb = idx_ref[0]
    ohcT = (idxb[0:1, :] == lax.broadcasted_iota(jnp.int32, (4, _BS), 0)
            ).astype(jnp.float32)
    ohfT = (idxb[1:2, :] == lax.broadcasted_iota(jnp.int32, (8, _BS), 0)
            ).astype(jnp.float32)

    W2T = W2T_ref[...]
    TcmsT = jnp.dot(W2T[:, 64:96], EcmsT_ref[...],
                    preferred_element_type=jnp.float32)
    TfnfT = jnp.dot(W2T[:, 96:128], EfnfT_ref[...],
                    preferred_element_type=jnp.float32)

    acc = jnp.dot(W2T[:, 0:64], f1T, preferred_element_type=jnp.float32)
    acc += jnp.dot(TcmsT, ohcT, preferred_element_type=jnp.float32)
    acc += jnp.dot(TfnfT, ohfT, preferred_element_type=jnp.float32)
    acc += lax.dot_general(W2T[:, 128:256], epc_ref[...],
                           (((1,), (1,)), ((), ())),
                           preferred_element_type=jnp.float32)
    o_ref[...] = jnp.tanh(acc + b2_ref[...])


def kernel(x, idx_cms, idx_fnf, idx_pc, W1, b1, E_cms, E_fnf, E_pc, W2, b2):
    e_pc = _sc_gather(E_pc, idx_pc.reshape(_B))

    idx2 = jnp.stack([idx_cms.reshape(_GRID, _BS),
                      idx_fnf.reshape(_GRID, _BS)], axis=1)
    xT = x.T
    W1T = W1.T
    W2T = W2.T
    EcmsT = E_cms.T
    EfnfT = jnp.zeros((32, 8), jnp.float32).at[:, :5].set(E_fnf.T)
    b1c = b1.reshape(64, 1)
    b2c = b2.reshape(64, 1)

    full = lambda i: (0, 0)
    yT = pl.pallas_call(
        _tc_body,
        grid=(_GRID,),
        in_specs=[
            pl.BlockSpec((3, _BS), lambda i: (0, i)),
            pl.BlockSpec((1, 2, _BS), lambda i: (i, 0, 0)),
            pl.BlockSpec((_BS, _D), lambda i: (i, 0)),
            pl.BlockSpec((64, 3), full),
            pl.BlockSpec((64, 1), full),
            pl.BlockSpec((32, 4), full),
            pl.BlockSpec((32, 8), full),
            pl.BlockSpec((64, 256), full),
            pl.BlockSpec((64, 1), full),
        ],
        out_specs=pl.BlockSpec((64, _BS), lambda i: (0, i)),
        out_shape=jax.ShapeDtypeStruct((64, _B), jnp.float32),
    )(xT, idx2, e_pc, W1T, b1c, EcmsT, EfnfT, W2T, b2c)
    return yT.T
